# baseline (speedup 1.0000x reference)
import functools

import jax
import jax.numpy as jnp
from jax import lax
from jax.experimental import pallas as pl
from jax.experimental.pallas import tpu as pltpu
from jax.experimental.pallas import tpu_sc as plsc

N = 10000
E = 320000
F = 128
H = 100
NC = 2
NS = 16
NW = NC * NS
NPAD = 10240
RPT = NPAD // NS
EPT = E // NW
NB = EPT // 16
CH = 125
NBUF = 5

_mesh = plsc.VectorSubcoreMesh(core_axis_name="c", subcore_axis_name="s")
_sc_params = pltpu.CompilerParams(
    needs_layout_passes=False, use_tc_tiling_on_sc=False)


@functools.partial(
    pl.kernel, mesh=_mesh,
    out_type=(jax.ShapeDtypeStruct((NC, NPAD, F), jnp.float32),
              jax.ShapeDtypeStruct((NPAD // 16, 16), jnp.float32)),
    compiler_params=_sc_params,
    scratch_types=[
        pltpu.VMEM((CH, 16), jnp.int32),
        pltpu.VMEM((CH, 16), jnp.int32),
        pltpu.VMEM((CH, 16), jnp.float32),
        pltpu.VMEM((NPAD // 16, 16), jnp.float32),
        pltpu.VMEM((NPAD // 16, 16), jnp.float32),
        pltpu.VMEM((NPAD // 16,), jnp.int32),
        pltpu.VMEM((NBUF, 16, F), jnp.float32),
        pltpu.VMEM((NBUF, 16, F), jnp.float32),
        pltpu.VMEM_SHARED((NPAD // 16, 16), jnp.float32),
        pltpu.VMEM_SHARED((NPAD, F), jnp.float32),
    ] + [pltpu.SemaphoreType.DMA] * (2 * NBUF))
def _sc_agg(src_hbm, dst_hbm, w_hbm, x_hbm, sp_hbm, dinv_hbm,
            src_b, dst_b, w_b, dinv2, degl2, iota_b, gbuf, sbuf,
            deg_sh, acc,
            g0, g1, g2, g3, g4, s0, s1, s2, s3, s4):
    gsem = (g0, g1, g2, g3, g4)
    ssem = (s0, s1, s2, s3, s4)
    c = lax.axis_index("c")
    s = lax.axis_index("s")
    wid = c * NS + s

    def z0(k, carry):
        degl2[k] = jnp.zeros((16,), jnp.float32)
        return carry
    lax.fori_loop(0, NPAD // 16, z0, 0)

    def z1(k, carry):
        iota_b[pl.ds(16 * k, 16)] = (
            lax.broadcasted_iota(jnp.int32, (16,), 0) + 16 * k)
        return carry
    lax.fori_loop(0, NPAD // 256, z1, 0)

    for r in range(16):
        for q in range(F // 16):
            gbuf[0, r, pl.ds(q * 16, 16)] = jnp.zeros((16,), jnp.float32)

    def zb(k, carry):
        pltpu.sync_copy(gbuf.at[0], acc.at[pl.ds(s * RPT + k * 16, 16)])
        return carry
    lax.fori_loop(0, RPT // 16, zb, 0)

    @pl.when(s == 0)
    def _():
        pltpu.sync_copy(degl2, deg_sh)
    plsc.subcore_barrier()

    def dchunk(h, carry):
        row = 2 * s + h // 5
        off = (h % 5) * CH
        pltpu.sync_copy(dst_hbm.at[row, pl.ds(off, CH)], dst_b)
        pltpu.sync_copy(w_hbm.at[row, pl.ds(off, CH)], w_b)

        def db(j, icarry):
            dv = dst_b[j]
            plsc.addupdate_scatter(
                degl2, [lax.shift_right_logical(dv, 4), dv & 15], w_b[j])
            return icarry
        lax.fori_loop(0, CH, db, 0)
        return carry
    lax.fori_loop(0, (E // 16) // NS // CH, dchunk, 0)

    pltpu.sync_copy(degl2, deg_sh.at[iota_b], add=True)
    plsc.subcore_barrier()

    pltpu.sync_copy(deg_sh, dinv2)

    def nwt(k, carry):
        d = dinv2[k] + 1.0
        i = plsc.bitcast(d, jnp.int32)
        y = plsc.bitcast(0x5F3759DF - lax.shift_right_logical(i, 1),
                         jnp.float32)
        for _ in range(3):
            y = y * (1.5 - 0.5 * d * y * y)
        dinv2[k] = y
        return carry
    lax.fori_loop(0, NPAD // 16, nwt, 0)

    @pl.when(c == 0)
    def _():
        dsl = pl.ds(s * (RPT // 16), RPT // 16)
        pltpu.sync_copy(dinv2.at[dsl], dinv_hbm.at[dsl])

    def chunk(ch, carry):
        csl = pl.ds(ch * CH, CH)
        pltpu.sync_copy(src_hbm.at[wid, csl], src_b)
        pltpu.sync_copy(dst_hbm.at[wid, csl], dst_b)
        pltpu.sync_copy(w_hbm.at[wid, csl], w_b)

        for b in range(NBUF):
            pltpu.make_async_copy(
                x_hbm.at[src_b.at[b]], gbuf.at[b], gsem[b]).start()

        def mb(i, icarry):
            for b in range(NBUF):
                j = i * NBUF + b
                pltpu.make_async_copy(
                    x_hbm.at[src_b.at[j]], gbuf.at[b], gsem[b]).wait()

                @pl.when(j >= NBUF)
                def _():
                    pltpu.make_async_copy(
                        sbuf.at[b], acc.at[dst_b.at[j]], ssem[b]).wait()
                sv = src_b[j]
                dv = dst_b[j]
                norm = (plsc.load_gather(
                            dinv2, [lax.shift_right_logical(sv, 4), sv & 15])
                        * w_b[j]
                        * plsc.load_gather(
                            dinv2, [lax.shift_right_logical(dv, 4), dv & 15]))
                for r in range(16):
                    sc = norm[r]
                    for q in range(F // 16):
                        sl2 = pl.ds(q * 16, 16)
                        sbuf[b, r, sl2] = gbuf[b, r, sl2] * sc
                pltpu.async_copy(sbuf.at[b], acc.at[dst_b.at[j]], ssem[b],
                                 add=True)
                nj = j + NBUF

                @pl.when(nj < CH)
                def _():
                    pltpu.make_async_copy(
                        x_hbm.at[src_b.at[nj]], gbuf.at[b], gsem[b]).start()
            return icarry
        lax.fori_loop(0, CH // NBUF, mb, 0)

        for b in range(NBUF):
            pltpu.make_async_copy(
                sbuf.at[b], acc.at[dst_b.at[CH - NBUF + b]], ssem[b]).wait()
        return carry
    lax.fori_loop(0, NB // CH, chunk, 0)

    plsc.subcore_barrier()
    sl = pl.ds(s * RPT, RPT)
    pltpu.sync_copy(acc.at[sl], sp_hbm.at[c, sl])


def _dense_body(sp_ref, x_ref, dv_ref, Wz_ref, Lzt_ref, lzb_ref, bz_ref,
                Wh_ref, Lht_ref, lhb_ref, bh_ref, lw_ref, lb_ref, o_ref):
    S = sp_ref[0] + sp_ref[1]
    dv = dv_ref[...]
    G = S + (dv * dv) * x_ref[...]
    Wzf = jnp.dot(Wz_ref[...], Lzt_ref[...], preferred_element_type=jnp.float32)
    Whf = jnp.dot(Wh_ref[...], Lht_ref[...], preferred_element_type=jnp.float32)
    bzf = jnp.dot(bz_ref[...], Lzt_ref[...], preferred_element_type=jnp.float32) + lzb_ref[...]
    bhf = jnp.dot(bh_ref[...], Lht_ref[...], preferred_element_type=jnp.float32) + lhb_ref[...]
    Z = jax.nn.sigmoid(jnp.dot(G, Wzf, preferred_element_type=jnp.float32) + bzf)
    T = jnp.tanh(jnp.dot(G, Whf, preferred_element_type=jnp.float32) + bhf)
    Hn = jnp.maximum((1.0 - Z) * T, 0.0)
    o_ref[...] = jnp.dot(Hn, lw_ref[...], preferred_element_type=jnp.float32) + lb_ref[...]


def kernel(x, edge_index, edge_weight, Wz, bz, Wr, br, Wh, bh,
           Lz_w, Lz_b, Lr_w, Lr_b, Lh_w, Lh_b, lin_w, lin_b):
    del Wr, br, Lr_w, Lr_b
    src2 = edge_index[0].reshape(NW, NB, 16)
    dst2 = edge_index[1].reshape(NW, NB, 16)
    w2 = edge_weight.reshape(NW, NB, 16)

    sp, dinv2 = _sc_agg(src2, dst2, w2, x)
    dinv = dinv2.reshape(NPAD)

    TM = 2000
    out = pl.pallas_call(
        _dense_body,
        grid=(N // TM,),
        in_specs=[
            pl.BlockSpec((NC, TM, F), lambda i: (0, i, 0)),
            pl.BlockSpec((TM, F), lambda i: (i, 0)),
            pl.BlockSpec((TM, 1), lambda i: (i, 0)),
            pl.BlockSpec((F, H), lambda i: (0, 0)),
            pl.BlockSpec((H, H), lambda i: (0, 0)),
            pl.BlockSpec((1, H), lambda i: (0, 0)),
            pl.BlockSpec((1, H), lambda i: (0, 0)),
            pl.BlockSpec((F, H), lambda i: (0, 0)),
            pl.BlockSpec((H, H), lambda i: (0, 0)),
            pl.BlockSpec((1, H), lambda i: (0, 0)),
            pl.BlockSpec((1, H), lambda i: (0, 0)),
            pl.BlockSpec((H, 1), lambda i: (0, 0)),
            pl.BlockSpec((1, 1), lambda i: (0, 0)),
        ],
        out_specs=pl.BlockSpec((TM, 1), lambda i: (i, 0)),
        out_shape=jax.ShapeDtypeStruct((N, 1), jnp.float32),
    )(sp, x, dinv[:N].reshape(N, 1),
      Wz, Lz_w[:H], Lz_b.reshape(1, H), bz.reshape(1, H),
      Wh, Lh_w[:H], Lh_b.reshape(1, H), bh.reshape(1, H),
      lin_w, lin_b.reshape(1, 1))
    return out


# speedup vs baseline: 1.1307x; 1.1307x over previous
import functools

import jax
import jax.numpy as jnp
from jax import lax
from jax.experimental import pallas as pl
from jax.experimental.pallas import tpu as pltpu
from jax.experimental.pallas import tpu_sc as plsc

N = 10000
E = 320000
F = 128
H = 100
NC = 2
NS = 16
NW = NC * NS
NPAD = 10240
RPT = NPAD // NS
EPT = E // NW
NB = EPT // 16
CH = 125
NBUF = 5

_mesh = plsc.VectorSubcoreMesh(core_axis_name="c", subcore_axis_name="s")
_sc_params = pltpu.CompilerParams(
    needs_layout_passes=False, use_tc_tiling_on_sc=False)


@functools.partial(
    pl.kernel, mesh=_mesh,
    out_type=jax.ShapeDtypeStruct((NW, 1, NPAD), jnp.float32),
    compiler_params=_sc_params,
    scratch_types=[
        pltpu.VMEM((NB, 16), jnp.int32),
        pltpu.VMEM((NB, 16), jnp.float32),
        pltpu.VMEM((NPAD,), jnp.float32),
    ])
def _sc_deg(dst_hbm, w_hbm, deg_hbm, dst_b, w_b, deg_l):
    c = lax.axis_index("c")
    s = lax.axis_index("s")
    wid = c * NS + s
    pltpu.sync_copy(dst_hbm.at[wid], dst_b)
    pltpu.sync_copy(w_hbm.at[wid], w_b)

    def zb(i, carry):
        deg_l[pl.ds(i * 16, 16)] = jnp.zeros((16,), jnp.float32)
        return carry
    lax.fori_loop(0, NPAD // 16, zb, 0)

    def eb(j, carry):
        plsc.addupdate_scatter(deg_l, [dst_b[j]], w_b[j])
        return carry
    lax.fori_loop(0, NB, eb, 0)

    pltpu.sync_copy(deg_l, deg_hbm.at[wid, 0])


def _dinv_body(dp_ref, o_ref):
    d = jnp.sum(dp_ref[...], axis=0) + 1.0
    o_ref[...] = lax.rsqrt(jnp.maximum(d, 1e-12))


@functools.partial(
    pl.kernel, mesh=_mesh,
    out_type=jax.ShapeDtypeStruct((NC, NPAD, F), jnp.float32),
    compiler_params=_sc_params,
    scratch_types=[
        pltpu.VMEM((CH, 16), jnp.int32),
        pltpu.VMEM((CH, 16), jnp.int32),
        pltpu.VMEM((CH, 16), jnp.float32),
        pltpu.VMEM((NPAD,), jnp.float32),
        pltpu.VMEM((NBUF, 16, F), jnp.float32),
        pltpu.VMEM((NBUF, 16, F), jnp.float32),
        pltpu.VMEM_SHARED((NPAD, F), jnp.float32),
    ] + [pltpu.SemaphoreType.DMA] * (2 * NBUF))
def _sc_agg(src_hbm, dst_hbm, w_hbm, dinv_hbm, x_hbm, sp_hbm,
            src_b, dst_b, w_b, dinv_t, gbuf, sbuf, acc,
            g0, g1, g2, g3, g4, s0, s1, s2, s3, s4):
    gsem = (g0, g1, g2, g3, g4)
    ssem = (s0, s1, s2, s3, s4)
    c = lax.axis_index("c")
    s = lax.axis_index("s")
    wid = c * NS + s
    pltpu.sync_copy(dinv_hbm, dinv_t)

    for r in range(16):
        for q in range(F // 16):
            gbuf[0, r, pl.ds(q * 16, 16)] = jnp.zeros((16,), jnp.float32)

    def zb(k, carry):
        pltpu.sync_copy(gbuf.at[0], acc.at[pl.ds(s * RPT + k * 16, 16)])
        return carry
    lax.fori_loop(0, RPT // 16, zb, 0)
    plsc.subcore_barrier()

    def chunk(ch, carry):
        csl = pl.ds(ch * CH, CH)
        pltpu.sync_copy(src_hbm.at[wid, csl], src_b)
        pltpu.sync_copy(dst_hbm.at[wid, csl], dst_b)
        pltpu.sync_copy(w_hbm.at[wid, csl], w_b)

        def gstart(j, b):
            for h in range(2):
                hs = pl.ds(h * 8, 8)
                pltpu.make_async_copy(
                    x_hbm.at[src_b.at[j, hs]], gbuf.at[b, hs], gsem[b]).start()

        def gwait(j, b):
            for h in range(2):
                hs = pl.ds(h * 8, 8)
                pltpu.make_async_copy(
                    x_hbm.at[src_b.at[j, hs]], gbuf.at[b, hs], gsem[b]).wait()

        for b in range(NBUF):
            gstart(b, b)

        def mb(i, icarry):
            for b in range(NBUF):
                j = i * NBUF + b
                gwait(j, b)

                @pl.when(j >= NBUF)
                def _():
                    pltpu.make_async_copy(
                        sbuf.at[b], acc.at[dst_b.at[j]], ssem[b]).wait()
                norm = (plsc.load_gather(dinv_t, [src_b[j]]) * w_b[j]
                        * plsc.load_gather(dinv_t, [dst_b[j]]))
                for r in range(16):
                    sc = norm[r]
                    for q in range(F // 16):
                        sl2 = pl.ds(q * 16, 16)
                        sbuf[b, r, sl2] = gbuf[b, r, sl2] * sc
                pltpu.async_copy(sbuf.at[b], acc.at[dst_b.at[j]], ssem[b],
                                 add=True)
                nj = j + NBUF

                @pl.when(nj < CH)
                def _():
                    gstart(nj, b)
            return icarry
        lax.fori_loop(0, CH // NBUF, mb, 0)

        for b in range(NBUF):
            pltpu.make_async_copy(
                sbuf.at[b], acc.at[dst_b.at[CH - NBUF + b]], ssem[b]).wait()
        return carry
    lax.fori_loop(0, NB // CH, chunk, 0)

    plsc.subcore_barrier()
    sl = pl.ds(s * RPT, RPT)
    pltpu.sync_copy(acc.at[sl], sp_hbm.at[c, sl])


def _dense_body(sp_ref, x_ref, dv_ref, Wz_ref, Lzt_ref, lzb_ref, bz_ref,
                Wh_ref, Lht_ref, lhb_ref, bh_ref, lw_ref, lb_ref, o_ref):
    S = sp_ref[0] + sp_ref[1]
    dv = dv_ref[...]
    G = S + (dv * dv) * x_ref[...]
    Wzf = jnp.dot(Wz_ref[...], Lzt_ref[...], preferred_element_type=jnp.float32)
    Whf = jnp.dot(Wh_ref[...], Lht_ref[...], preferred_element_type=jnp.float32)
    bzf = jnp.dot(bz_ref[...], Lzt_ref[...], preferred_element_type=jnp.float32) + lzb_ref[...]
    bhf = jnp.dot(bh_ref[...], Lht_ref[...], preferred_element_type=jnp.float32) + lhb_ref[...]
    Z = jax.nn.sigmoid(jnp.dot(G, Wzf, preferred_element_type=jnp.float32) + bzf)
    T = jnp.tanh(jnp.dot(G, Whf, preferred_element_type=jnp.float32) + bhf)
    Hn = jnp.maximum((1.0 - Z) * T, 0.0)
    o_ref[...] = jnp.dot(Hn, lw_ref[...], preferred_element_type=jnp.float32) + lb_ref[...]


def kernel(x, edge_index, edge_weight, Wz, bz, Wr, br, Wh, bh,
           Lz_w, Lz_b, Lr_w, Lr_b, Lh_w, Lh_b, lin_w, lin_b):
    del Wr, br, Lr_w, Lr_b
    src2 = edge_index[0].reshape(NW, NB, 16)
    dst2 = edge_index[1].reshape(NW, NB, 16)
    w2 = edge_weight.reshape(NW, NB, 16)

    deg_p = _sc_deg(dst2, w2)
    dinv = pl.pallas_call(
        _dinv_body,
        out_shape=jax.ShapeDtypeStruct((NPAD // 128, 128), jnp.float32),
    )(deg_p.reshape(NW, NPAD // 128, 128))
    dinv = dinv.reshape(NPAD)

    sp = _sc_agg(src2, dst2, w2, dinv, x)

    TM = 2000
    out = pl.pallas_call(
        _dense_body,
        grid=(N // TM,),
        in_specs=[
            pl.BlockSpec((NC, TM, F), lambda i: (0, i, 0)),
            pl.BlockSpec((TM, F), lambda i: (i, 0)),
            pl.BlockSpec((TM, 1), lambda i: (i, 0)),
            pl.BlockSpec((F, H), lambda i: (0, 0)),
            pl.BlockSpec((H, H), lambda i: (0, 0)),
            pl.BlockSpec((1, H), lambda i: (0, 0)),
            pl.BlockSpec((1, H), lambda i: (0, 0)),
            pl.BlockSpec((F, H), lambda i: (0, 0)),
            pl.BlockSpec((H, H), lambda i: (0, 0)),
            pl.BlockSpec((1, H), lambda i: (0, 0)),
            pl.BlockSpec((1, H), lambda i: (0, 0)),
            pl.BlockSpec((H, 1), lambda i: (0, 0)),
            pl.BlockSpec((1, 1), lambda i: (0, 0)),
        ],
        out_specs=pl.BlockSpec((TM, 1), lambda i: (i, 0)),
        out_shape=jax.ShapeDtypeStruct((N, 1), jnp.float32),
    )(sp, x, dinv[:N].reshape(N, 1),
      Wz, Lz_w[:H], Lz_b.reshape(1, H), bz.reshape(1, H),
      Wh, Lh_w[:H], Lh_b.reshape(1, H), bh.reshape(1, H),
      lin_w, lin_b.reshape(1, 1))
    return out


# speedup vs baseline: 1.1879x; 1.0506x over previous
import functools

import jax
import jax.numpy as jnp
from jax import lax
from jax.experimental import pallas as pl
from jax.experimental.pallas import tpu as pltpu
from jax.experimental.pallas import tpu_sc as plsc

N = 10000
E = 320000
F = 128
H = 100
NC = 2
NS = 16
NW = NC * NS
NPAD = 10240
RPT = NPAD // NS
EPT = E // NW
NB = EPT // 16
CH = 125
NBUF = 5

_mesh = plsc.VectorSubcoreMesh(core_axis_name="c", subcore_axis_name="s")
_sc_params = pltpu.CompilerParams(
    needs_layout_passes=False, use_tc_tiling_on_sc=False)


@functools.partial(
    pl.kernel, mesh=_mesh,
    out_type=jax.ShapeDtypeStruct((NW, 1, NPAD), jnp.float32),
    compiler_params=_sc_params,
    scratch_types=[
        pltpu.VMEM((NB, 16), jnp.int32),
        pltpu.VMEM((NB, 16), jnp.float32),
        pltpu.VMEM((NPAD,), jnp.float32),
    ])
def _sc_deg(dst_hbm, w_hbm, deg_hbm, dst_b, w_b, deg_l):
    c = lax.axis_index("c")
    s = lax.axis_index("s")
    wid = c * NS + s
    pltpu.sync_copy(dst_hbm.at[wid], dst_b)
    pltpu.sync_copy(w_hbm.at[wid], w_b)

    def zb(i, carry):
        deg_l[pl.ds(i * 16, 16)] = jnp.zeros((16,), jnp.float32)
        return carry
    lax.fori_loop(0, NPAD // 16, zb, 0)

    def eb(j, carry):
        plsc.addupdate_scatter(deg_l, [dst_b[j]], w_b[j])
        return carry
    lax.fori_loop(0, NB, eb, 0)

    pltpu.sync_copy(deg_l, deg_hbm.at[wid, 0])


def _dinv_body(dp_ref, o_ref):
    d = jnp.sum(dp_ref[...], axis=0) + 1.0
    o_ref[...] = lax.rsqrt(jnp.maximum(d, 1e-12))


@functools.partial(
    pl.kernel, mesh=_mesh,
    out_type=jax.ShapeDtypeStruct((NC, NPAD, F), jnp.float32),
    compiler_params=_sc_params,
    scratch_types=[
        pltpu.VMEM((CH, 16), jnp.int32),
        pltpu.VMEM((CH, 16), jnp.int32),
        pltpu.VMEM((CH, 16), jnp.float32),
        pltpu.VMEM((NPAD,), jnp.float32),
        pltpu.VMEM((NBUF, 16, F), jnp.bfloat16),
        pltpu.VMEM((NBUF, 16, F), jnp.float32),
        pltpu.VMEM_SHARED((NPAD, F), jnp.float32),
    ] + [pltpu.SemaphoreType.DMA] * (2 * NBUF))
def _sc_agg(src_hbm, dst_hbm, w_hbm, dinv_hbm, x_hbm, sp_hbm,
            src_b, dst_b, w_b, dinv_t, gbuf, sbuf, acc,
            g0, g1, g2, g3, g4, s0, s1, s2, s3, s4):
    gsem = (g0, g1, g2, g3, g4)
    ssem = (s0, s1, s2, s3, s4)
    c = lax.axis_index("c")
    s = lax.axis_index("s")
    wid = c * NS + s
    pltpu.sync_copy(dinv_hbm, dinv_t)

    for r in range(16):
        for q in range(F // 16):
            sbuf[0, r, pl.ds(q * 16, 16)] = jnp.zeros((16,), jnp.float32)

    def zb(k, carry):
        pltpu.sync_copy(sbuf.at[0], acc.at[pl.ds(s * RPT + k * 16, 16)])
        return carry
    lax.fori_loop(0, RPT // 16, zb, 0)
    plsc.subcore_barrier()

    def chunk(ch, carry):
        csl = pl.ds(ch * CH, CH)
        pltpu.sync_copy(src_hbm.at[wid, csl], src_b)
        pltpu.sync_copy(dst_hbm.at[wid, csl], dst_b)
        pltpu.sync_copy(w_hbm.at[wid, csl], w_b)

        def gstart(j, b):
            for h in range(2):
                hs = pl.ds(h * 8, 8)
                pltpu.make_async_copy(
                    x_hbm.at[src_b.at[j, hs]], gbuf.at[b, hs], gsem[b]).start()

        def gwait(j, b):
            for h in range(2):
                hs = pl.ds(h * 8, 8)
                pltpu.make_async_copy(
                    x_hbm.at[src_b.at[j, hs]], gbuf.at[b, hs], gsem[b]).wait()

        for b in range(NBUF):
            gstart(b, b)

        def mb(i, icarry):
            for b in range(NBUF):
                j = i * NBUF + b
                gwait(j, b)

                @pl.when(j >= NBUF)
                def _():
                    pltpu.make_async_copy(
                        sbuf.at[b], acc.at[dst_b.at[j]], ssem[b]).wait()
                norm = (plsc.load_gather(dinv_t, [src_b[j]]) * w_b[j]
                        * plsc.load_gather(dinv_t, [dst_b[j]]))
                for r in range(16):
                    sc = norm[r]
                    for q in range(F // 32):
                        v = gbuf[b, r, pl.ds(q * 32, 32)]
                        lo, hi = plsc.unpack(
                            v, format=plsc.PackFormat.INTERLEAVED,
                            preferred_element_type=jnp.float32)
                        sbuf[b, r, pl.ds(q * 32, 16)] = lo * sc
                        sbuf[b, r, pl.ds(q * 32 + 16, 16)] = hi * sc
                pltpu.async_copy(sbuf.at[b], acc.at[dst_b.at[j]], ssem[b],
                                 add=True)
                nj = j + NBUF

                @pl.when(nj < CH)
                def _():
                    gstart(nj, b)
            return icarry
        lax.fori_loop(0, CH // NBUF, mb, 0)

        for b in range(NBUF):
            pltpu.make_async_copy(
                sbuf.at[b], acc.at[dst_b.at[CH - NBUF + b]], ssem[b]).wait()
        return carry
    lax.fori_loop(0, NB // CH, chunk, 0)

    plsc.subcore_barrier()
    sl = pl.ds(s * RPT, RPT)
    pltpu.sync_copy(acc.at[sl], sp_hbm.at[c, sl])


def _dense_body(sp_ref, x_ref, dv_ref, Wz_ref, Lzt_ref, lzb_ref, bz_ref,
                Wh_ref, Lht_ref, lhb_ref, bh_ref, lw_ref, lb_ref, o_ref):
    S = sp_ref[0] + sp_ref[1]
    dv = dv_ref[...]
    G = S + (dv * dv) * x_ref[...]
    Wzf = jnp.dot(Wz_ref[...], Lzt_ref[...], preferred_element_type=jnp.float32)
    Whf = jnp.dot(Wh_ref[...], Lht_ref[...], preferred_element_type=jnp.float32)
    bzf = jnp.dot(bz_ref[...], Lzt_ref[...], preferred_element_type=jnp.float32) + lzb_ref[...]
    bhf = jnp.dot(bh_ref[...], Lht_ref[...], preferred_element_type=jnp.float32) + lhb_ref[...]
    Z = jax.nn.sigmoid(jnp.dot(G, Wzf, preferred_element_type=jnp.float32) + bzf)
    T = jnp.tanh(jnp.dot(G, Whf, preferred_element_type=jnp.float32) + bhf)
    Hn = jnp.maximum((1.0 - Z) * T, 0.0)
    o_ref[...] = jnp.dot(Hn, lw_ref[...], preferred_element_type=jnp.float32) + lb_ref[...]


def kernel(x, edge_index, edge_weight, Wz, bz, Wr, br, Wh, bh,
           Lz_w, Lz_b, Lr_w, Lr_b, Lh_w, Lh_b, lin_w, lin_b):
    del Wr, br, Lr_w, Lr_b
    src2 = edge_index[0].reshape(NW, NB, 16)
    dst2 = edge_index[1].reshape(NW, NB, 16)
    w2 = edge_weight.reshape(NW, NB, 16)

    deg_p = _sc_deg(dst2, w2)
    dinv = pl.pallas_call(
        _dinv_body,
        out_shape=jax.ShapeDtypeStruct((NPAD // 128, 128), jnp.float32),
    )(deg_p.reshape(NW, NPAD // 128, 128))
    dinv = dinv.reshape(NPAD)

    xs = (x.astype(jnp.bfloat16).reshape(N, F // 32, 2, 16)
          .transpose(0, 1, 3, 2).reshape(N, F))
    sp = _sc_agg(src2, dst2, w2, dinv, xs)

    TM = 2000
    out = pl.pallas_call(
        _dense_body,
        grid=(N // TM,),
        in_specs=[
            pl.BlockSpec((NC, TM, F), lambda i: (0, i, 0)),
            pl.BlockSpec((TM, F), lambda i: (i, 0)),
            pl.BlockSpec((TM, 1), lambda i: (i, 0)),
            pl.BlockSpec((F, H), lambda i: (0, 0)),
            pl.BlockSpec((H, H), lambda i: (0, 0)),
            pl.BlockSpec((1, H), lambda i: (0, 0)),
            pl.BlockSpec((1, H), lambda i: (0, 0)),
            pl.BlockSpec((F, H), lambda i: (0, 0)),
            pl.BlockSpec((H, H), lambda i: (0, 0)),
            pl.BlockSpec((1, H), lambda i: (0, 0)),
            pl.BlockSpec((1, H), lambda i: (0, 0)),
            pl.BlockSpec((H, 1), lambda i: (0, 0)),
            pl.BlockSpec((1, 1), lambda i: (0, 0)),
        ],
        out_specs=pl.BlockSpec((TM, 1), lambda i: (i, 0)),
        out_shape=jax.ShapeDtypeStruct((N, 1), jnp.float32),
    )(sp, x, dinv[:N].reshape(N, 1),
      Wz, Lz_w[:H], Lz_b.reshape(1, H), bz.reshape(1, H),
      Wh, Lh_w[:H], Lh_b.reshape(1, H), bh.reshape(1, H),
      lin_w, lin_b.reshape(1, 1))
    return out
